# baseline (speedup 1.0000x reference)
import functools
import math

import jax
import jax.numpy as jnp
from jax import lax
from jax.experimental import pallas as pl
from jax.experimental.pallas import tpu as pltpu
from jax.experimental.pallas import tpu_sc as plsc

DM = 64
SCALE = math.sqrt(DM)

NC = 2
NS = 16
NW = NC * NS
L = 16
BW = 128
CS = 2
PADW = BW + 1


def _emb_lookup(table, xt):
    sl, nb = xt.shape
    assert nb == NW * BW
    n_chunks = sl // CS
    assert sl % CS == 0 and n_chunks % 2 == 0

    mesh = plsc.VectorSubcoreMesh(core_axis_name="c", subcore_axis_name="s")

    @functools.partial(
        pl.kernel,
        mesh=mesh,
        out_type=jax.ShapeDtypeStruct((sl, DM, nb), jnp.float32),
        scratch_types=[
            pltpu.VMEM((2, CS, BW), jnp.int32),
            pltpu.VMEM((2, CS * BW, DM), jnp.float32),
            pltpu.VMEM((2, CS, DM, PADW), jnp.float32),
            pltpu.SemaphoreType.DMA,
            pltpu.SemaphoreType.DMA,
            pltpu.SemaphoreType.DMA,
            pltpu.SemaphoreType.DMA,
        ],
        compiler_params=pltpu.CompilerParams(use_tc_tiling_on_sc=False,
                                             needs_layout_passes=False),
    )
    def k(table_hbm, xt_hbm, out_hbm, idx_v, g_v, o_v, gsem0, gsem1,
          wsem0, wsem1):
        gsems = (gsem0, gsem1)
        wsems = (wsem0, wsem1)
        wid = lax.axis_index("s") * NC + lax.axis_index("c")
        b0 = wid * BW

        def fire(c, bb):
            s0 = c * CS
            pltpu.sync_copy(xt_hbm.at[pl.ds(s0, CS), pl.ds(b0, BW)],
                            idx_v.at[bb])
            for r in range(CS):
                pltpu.async_copy(
                    table_hbm.at[idx_v.at[bb, r]],
                    g_v.at[bb, pl.ds(r * BW, BW)],
                    gsems[bb],
                )

        def drain_g(bb):
            pltpu.make_async_copy(table_hbm.at[pl.ds(0, CS * BW), :],
                                  g_v.at[bb], gsems[bb]).wait()

        def drain_w(bb):
            pltpu.make_async_copy(
                out_hbm.at[pl.ds(0, CS), :, pl.ds(0, BW)],
                o_v.at[bb, :, :, pl.ds(0, BW)], wsems[bb]).wait()

        fire(0, 0)
        rows0 = [lax.iota(jnp.int32, L) + j * L for j in range(DM // L)]

        def pair(t, carry):
            go = t * 2
            for b in (0, 1):
                c = go + b
                nb_ = 1 - b

                @pl.when(c + 1 < n_chunks)
                def _():
                    @pl.when(c >= 1)
                    def _():
                        drain_w(nb_, )
                    fire(c + 1, nb_)

                drain_g(b)

                def repack(kk, cr, _b=b):
                    s_l = lax.shift_right_logical(kk, 7)
                    b_l = lax.bitwise_and(kk, BW - 1)
                    cols = jnp.full((L,), b_l, jnp.int32)
                    dst = o_v.at[_b, s_l]
                    for j in range(DM // L):
                        v = g_v[_b, kk, pl.ds(j * L, L)] * SCALE
                        plsc.store_scatter(dst, [rows0[j], cols], v)
                    return cr

                lax.fori_loop(0, CS * BW, repack, 0, unroll=2)
                pltpu.async_copy(
                    o_v.at[b, :, :, pl.ds(0, BW)],
                    out_hbm.at[pl.ds(c * CS, CS), :, pl.ds(b0, BW)],
                    wsems[b],
                )
            return carry

        lax.fori_loop(0, n_chunks // 2, pair, 0)
        drain_w(0)
        drain_w(1)

    return k(table, xt)


def kernel(x, table):
    out = _emb_lookup(table, x.T)
    return jnp.transpose(out, (2, 0, 1))


# speedup vs baseline: 1.1450x; 1.1450x over previous
import functools
import math

import jax
import jax.numpy as jnp
from jax import lax
from jax.experimental import pallas as pl
from jax.experimental.pallas import tpu as pltpu
from jax.experimental.pallas import tpu_sc as plsc

DM = 64
SCALE = math.sqrt(DM)

NC = 2
NS = 16
NW = NC * NS
L = 16

CB = 4


def _emb_lookup(table, x):
    nb, sl = x.shape
    rows_per_w = nb // NW
    n_chunks = rows_per_w // CB
    assert rows_per_w % CB == 0 and n_chunks % 2 == 0
    g0 = 128
    g1 = sl - g0

    mesh = plsc.VectorSubcoreMesh(core_axis_name="c", subcore_axis_name="s")

    @functools.partial(
        pl.kernel,
        mesh=mesh,
        out_type=jax.ShapeDtypeStruct((nb, sl, DM), jnp.float32),
        scratch_types=[
            pltpu.VMEM((2, CB, sl), jnp.int32),
            pltpu.VMEM((2, CB, sl, DM), jnp.float32),
            pltpu.SemaphoreType.DMA,
            pltpu.SemaphoreType.DMA,
            pltpu.SemaphoreType.DMA,
            pltpu.SemaphoreType.DMA,
        ],
        compiler_params=pltpu.CompilerParams(use_tc_tiling_on_sc=False),
    )
    def k(table_hbm, x_hbm, out_hbm, idx_v, rows_v, gsem0, gsem1, wsem0,
          wsem1):
        gsems = (gsem0, gsem1)
        wsems = (wsem0, wsem1)
        wid = lax.axis_index("s") * NC + lax.axis_index("c")
        base = wid * rows_per_w

        def fire(c, bb):
            b0 = base + c * CB
            pltpu.sync_copy(x_hbm.at[pl.ds(b0, CB)], idx_v.at[bb])
            for r in range(CB):
                pltpu.async_copy(
                    table_hbm.at[idx_v.at[bb, r, pl.ds(0, g0)]],
                    rows_v.at[bb, r, pl.ds(0, g0)],
                    gsems[bb],
                )
                pltpu.async_copy(
                    table_hbm.at[idx_v.at[bb, r, pl.ds(g0, g1)]],
                    rows_v.at[bb, r, pl.ds(g0, g1)],
                    gsems[bb],
                )

        def drain(sem, bb):
            pltpu.make_async_copy(out_hbm.at[pl.ds(0, CB)],
                                  rows_v.at[bb], sem).wait()

        fire(0, 0)

        def pair(t, carry):
            go = t * 2
            for b in (0, 1):
                c = go + b
                nb_ = 1 - b

                @pl.when(c + 1 < n_chunks)
                def _():
                    @pl.when(c >= 1)
                    def _():
                        drain(wsems[nb_], nb_)
                    fire(c + 1, nb_)

                drain(gsems[b], b)

                for r in range(CB):

                    def scale_row(i, cr, _b=b, _r=r):
                        for j in range(DM // L):
                            s = pl.ds(j * L, L)
                            rows_v[_b, _r, i, s] = rows_v[_b, _r, i, s] * SCALE
                        return cr

                    lax.fori_loop(0, sl, scale_row, 0, unroll=4)

                pltpu.async_copy(
                    rows_v.at[b],
                    out_hbm.at[pl.ds(base + c * CB, CB)],
                    wsems[b],
                )
            return carry

        lax.fori_loop(0, n_chunks // 2, pair, 0)
        drain(wsems[0], 0)
        drain(wsems[1], 1)

    return k(table, x)


def kernel(x, table):
    return _emb_lookup(table, x)
